# baseline (speedup 1.0000x reference)
import jax
import jax.numpy as jnp
from jax import lax
from jax.experimental import pallas as pl
from jax.experimental.pallas import tpu as pltpu
from jax.experimental.pallas import tpu_sc as plsc

N = 320000
D = 128
K = 1024
RB = 128
BLK = 256
NBLK = N // BLK
NW = 32
BASE_BLKS = NBLK // NW
EXTRA = NBLK - BASE_BLKS * NW
NRING = 3
TRIPS = BASE_BLKS // NRING


def _sc_body(emb, lbl, part_o, cnt_o, sq_o, acc_sp,
             buf0, buf1, buf2, la0, lb0, la1, lb1, la2, lb2, cnt, cntred,
             sqbuf, sin0, sin1, sin2, sl0, sl1, sl2, ssc0, ssc1, ssc2):
    c = lax.axis_index("c")
    s = lax.axis_index("s")
    wid = s * 2 + c

    zf16 = jnp.zeros((16,), jnp.float32)
    zi16 = jnp.zeros((16,), jnp.int32)
    iota16 = lax.broadcasted_iota(jnp.int32, (16,), 0)
    ones16 = jnp.ones((16,), jnp.int32)
    bufs = (buf0, buf1, buf2)
    las = (la0, la1, la2)
    lbs = (lb0, lb1, lb2)
    sins = (sin0, sin1, sin2)
    sls = (sl0, sl1, sl2)
    sscs = (ssc0, ssc1, ssc2)

    def zero_cnt(i, _):
        for j in range(16):
            cnt[j, pl.ds(i * 16, 16)] = zi16
        return 0
    lax.fori_loop(0, K // 16, zero_cnt, 0)
    sqbuf[pl.ds(0, 16)] = zf16

    def zero_buf(i, _):
        for v in range(8):
            buf0[i, pl.ds(v * 16, 16)] = zf16
        return 0
    lax.fori_loop(0, RB, zero_buf, 0)

    pltpu.sync_copy(buf0.at[pl.ds(0, K // 16)],
                    acc_sp.at[pl.ds(s * (K // 16), K // 16)])

    plsc.subcore_barrier()

    start = wid * BASE_BLKS

    def start_in(blkidx, p):
        row0 = blkidx * BLK
        pltpu.async_copy(emb.at[pl.ds(row0, BLK)], bufs[p], sins[p])
        pltpu.async_copy(lbl.at[pl.ds(row0, RB)], las[p], sls[p])
        pltpu.async_copy(lbl.at[pl.ds(row0 + RB, RB)], lbs[p], sls[p])

    def wait_in(p):
        pltpu.make_async_copy(emb.at[pl.ds(0, BLK)], bufs[p], sins[p]).wait()
        pltpu.make_async_copy(lbl.at[pl.ds(0, RB)], las[p], sls[p]).wait()
        pltpu.make_async_copy(lbl.at[pl.ds(0, RB)], lbs[p], sls[p]).wait()

    def start_scat(p):
        da = pltpu.async_copy(bufs[p].at[pl.ds(0, RB)],
                              acc_sp.at[las[p]], sscs[p], add=True)
        db = pltpu.async_copy(bufs[p].at[pl.ds(RB, RB)],
                              acc_sp.at[lbs[p]], sscs[p], add=True)
        return da, db

    def compute(p):
        bf = bufs[p]
        for lb_ref in (las[p], lbs[p]):
            for t in range(8):
                l16 = lb_ref[pl.ds(t * 16, 16)]
                plsc.addupdate_scatter(cnt, [iota16, l16], ones16)

        def srow(r, a):
            for v in range(8):
                x0 = bf[2 * r, pl.ds(v * 16, 16)]
                x1 = bf[2 * r + 1, pl.ds(v * 16, 16)]
                a = a + x0 * x0 + x1 * x1
            return a
        blocksq = lax.fori_loop(0, BLK // 2, srow, jnp.zeros((16,), jnp.float32))
        sqbuf[pl.ds(0, 16)] = sqbuf[pl.ds(0, 16)] + blocksq

    for p in range(NRING):
        start_in(start + p, p)

    def tri_body(i, _):
        b0 = start + NRING * i
        for p in range(NRING):
            wait_in(p)
            d = start_scat(p)
            compute(p)
            d[0].wait()
            d[1].wait()

            @pl.when(i < TRIPS - 1)
            def _():
                start_in(b0 + p + NRING, p)
        return 0

    lax.fori_loop(0, TRIPS, tri_body, 0)

    @pl.when(wid < EXTRA)
    def _():
        start_in(NW * BASE_BLKS + wid, 0)
        wait_in(0)
        pltpu.sync_copy(buf0.at[pl.ds(0, RB)], acc_sp.at[la0], add=True)
        pltpu.sync_copy(buf0.at[pl.ds(RB, RB)], acc_sp.at[lb0], add=True)
        compute(0)

    def red_cnt(g, _):
        a = cnt[0, pl.ds(g * 16, 16)]
        for j in range(1, 16):
            a = a + cnt[j, pl.ds(g * 16, 16)]
        cntred[pl.ds(g * 16, 16)] = a
        return 0
    lax.fori_loop(0, K // 16, red_cnt, 0)

    pltpu.sync_copy(cntred, cnt_o.at[wid])
    pltpu.sync_copy(sqbuf, sq_o.at[wid])

    plsc.subcore_barrier()

    pltpu.sync_copy(acc_sp.at[pl.ds(s * (K // 16), K // 16)],
                    part_o.at[c, pl.ds(s * (K // 16), K // 16)])


def _epi_body(part_ref, cnt_ref, sq_ref, out_ref):
    S = part_ref[0] + part_ref[1]
    rowsq = jnp.sum(S * S, axis=1)
    tot = jnp.sum(S, axis=0)
    tot2 = jnp.sum(tot * tot)
    counts = jnp.sum(cnt_ref[...], axis=0)
    countsf = counts.astype(jnp.float32)
    present = counts > 0
    k = jnp.sum(present.astype(jnp.int32))
    safe = jnp.where(present, countsf, jnp.float32(1.0))
    T = jnp.sum(rowsq / safe)
    sumsq = jnp.sum(sq_ref[...])
    n = jnp.float32(N)
    bcss = T - tot2 / n
    wcss = sumsq - T
    kf = k.astype(jnp.float32)
    ch = bcss * (n - kf) / ((kf - 1.0) * wcss + jnp.float32(1e-10))
    val = jnp.where((k < 2) | (k == N), jnp.float32(0.0), -ch)
    out_ref[...] = jnp.broadcast_to(val, (1, 1))


def kernel(embeddings, labels):
    labels = labels.reshape(-1)
    mesh = plsc.VectorSubcoreMesh(core_axis_name="c", subcore_axis_name="s")
    part, cnt, sq = pl.kernel(
        _sc_body,
        out_type=(
            jax.ShapeDtypeStruct((2, K, D), jnp.float32),
            jax.ShapeDtypeStruct((NW, K), jnp.int32),
            jax.ShapeDtypeStruct((NW, 16), jnp.float32),
        ),
        mesh=mesh,
        compiler_params=pltpu.CompilerParams(needs_layout_passes=False),
        scratch_types=[
            pltpu.VMEM_SHARED((K, D), jnp.float32),
            pltpu.VMEM((BLK, D), jnp.float32),
            pltpu.VMEM((BLK, D), jnp.float32),
            pltpu.VMEM((BLK, D), jnp.float32),
            pltpu.VMEM((RB,), jnp.int32),
            pltpu.VMEM((RB,), jnp.int32),
            pltpu.VMEM((RB,), jnp.int32),
            pltpu.VMEM((RB,), jnp.int32),
            pltpu.VMEM((RB,), jnp.int32),
            pltpu.VMEM((RB,), jnp.int32),
            pltpu.VMEM((16, K), jnp.int32),
            pltpu.VMEM((K,), jnp.int32),
            pltpu.VMEM((16,), jnp.float32),
            pltpu.SemaphoreType.DMA,
            pltpu.SemaphoreType.DMA,
            pltpu.SemaphoreType.DMA,
            pltpu.SemaphoreType.DMA,
            pltpu.SemaphoreType.DMA,
            pltpu.SemaphoreType.DMA,
            pltpu.SemaphoreType.DMA,
            pltpu.SemaphoreType.DMA,
            pltpu.SemaphoreType.DMA,
        ],
    )(embeddings, labels)
    return part[0, 0, 0] + sq[0, 0] + cnt[0, 0].astype(jnp.float32)


# speedup vs baseline: 1.0261x; 1.0261x over previous
import jax
import jax.numpy as jnp
from jax import lax
from jax.experimental import pallas as pl
from jax.experimental.pallas import tpu as pltpu
from jax.experimental.pallas import tpu_sc as plsc

N = 320000
D = 128
K = 1024
RB = 128
BLK = 256
NBLK = N // BLK
NW = 32
BASE_BLKS = NBLK // NW
EXTRA = NBLK - BASE_BLKS * NW
NRING = 3
TRIPS = BASE_BLKS // NRING


def _sc_body(emb, lbl, part_o, cnt_o, sq_o, acc_sp,
             buf0, buf1, buf2, la0, lb0, la1, lb1, la2, lb2, cnt, cntred,
             sqbuf, sin0, sin1, sin2, sl0, sl1, sl2, ssc0, ssc1, ssc2):
    c = lax.axis_index("c")
    s = lax.axis_index("s")
    wid = s * 2 + c

    zf16 = jnp.zeros((16,), jnp.float32)
    zi16 = jnp.zeros((16,), jnp.int32)
    iota16 = lax.broadcasted_iota(jnp.int32, (16,), 0)
    ones16 = jnp.ones((16,), jnp.int32)
    bufs = (buf0, buf1, buf2)
    las = (la0, la1, la2)
    lbs = (lb0, lb1, lb2)
    sins = (sin0, sin1, sin2)
    sls = (sl0, sl1, sl2)
    sscs = (ssc0, ssc1, ssc2)

    def zero_cnt(i, _):
        for j in range(16):
            cnt[j, pl.ds(i * 16, 16)] = zi16
        return 0
    lax.fori_loop(0, K // 16, zero_cnt, 0)
    sqbuf[pl.ds(0, 16)] = zf16

    def zero_buf(i, _):
        for v in range(8):
            buf0[i, pl.ds(v * 16, 16)] = zf16
        return 0
    lax.fori_loop(0, RB, zero_buf, 0)

    pltpu.sync_copy(buf0.at[pl.ds(0, K // 16)],
                    acc_sp.at[pl.ds(s * (K // 16), K // 16)])

    plsc.subcore_barrier()

    start = wid * BASE_BLKS

    def start_in(blkidx, p):
        row0 = blkidx * BLK
        pltpu.async_copy(emb.at[pl.ds(row0, BLK)], bufs[p], sins[p])
        pltpu.async_copy(lbl.at[pl.ds(row0, RB)], las[p], sls[p])
        pltpu.async_copy(lbl.at[pl.ds(row0 + RB, RB)], lbs[p], sls[p])

    def wait_in(p):
        pltpu.make_async_copy(emb.at[pl.ds(0, BLK)], bufs[p], sins[p]).wait()
        pltpu.make_async_copy(lbl.at[pl.ds(0, RB)], las[p], sls[p]).wait()
        pltpu.make_async_copy(lbl.at[pl.ds(0, RB)], lbs[p], sls[p]).wait()

    def start_scat(p):
        da = pltpu.async_copy(bufs[p].at[pl.ds(0, RB)],
                              acc_sp.at[las[p]], sscs[p], add=True)
        db = pltpu.async_copy(bufs[p].at[pl.ds(RB, RB)],
                              acc_sp.at[lbs[p]], sscs[p], add=True)
        return da, db

    def compute(p):
        bf = bufs[p]
        for lb_ref in (las[p], lbs[p]):
            for t in range(8):
                l16 = lb_ref[pl.ds(t * 16, 16)]
                plsc.addupdate_scatter(cnt, [iota16, l16], ones16)

        def srow(r, a):
            for v in range(8):
                x0 = bf[2 * r, pl.ds(v * 16, 16)]
                x1 = bf[2 * r + 1, pl.ds(v * 16, 16)]
                a = a + x0 * x0 + x1 * x1
            return a
        blocksq = lax.fori_loop(0, BLK // 2, srow, jnp.zeros((16,), jnp.float32))
        sqbuf[pl.ds(0, 16)] = sqbuf[pl.ds(0, 16)] + blocksq

    for p in range(NRING):
        start_in(start + p, p)

    def tri_body(i, _):
        b0 = start + NRING * i
        for p in range(NRING):
            wait_in(p)
            d = start_scat(p)
            compute(p)
            d[0].wait()
            d[1].wait()

            @pl.when(i < TRIPS - 1)
            def _():
                start_in(b0 + p + NRING, p)
        return 0

    lax.fori_loop(0, TRIPS, tri_body, 0)

    @pl.when(wid < EXTRA)
    def _():
        start_in(NW * BASE_BLKS + wid, 0)
        wait_in(0)
        pltpu.sync_copy(buf0.at[pl.ds(0, RB)], acc_sp.at[la0], add=True)
        pltpu.sync_copy(buf0.at[pl.ds(RB, RB)], acc_sp.at[lb0], add=True)
        compute(0)

    def red_cnt(g, _):
        a = cnt[0, pl.ds(g * 16, 16)]
        for j in range(1, 16):
            a = a + cnt[j, pl.ds(g * 16, 16)]
        cntred[pl.ds(g * 16, 16)] = a
        return 0
    lax.fori_loop(0, K // 16, red_cnt, 0)

    pltpu.sync_copy(cntred, cnt_o.at[wid])
    pltpu.sync_copy(sqbuf, sq_o.at[wid])

    plsc.subcore_barrier()

    pltpu.sync_copy(acc_sp.at[pl.ds(s * (K // 16), K // 16)],
                    part_o.at[c, pl.ds(s * (K // 16), K // 16)])


def _epi_body(part_ref, cnt_ref, sq_ref, out_ref):
    S = part_ref[0] + part_ref[1]
    rowsq = jnp.sum(S * S, axis=1)
    tot = jnp.sum(S, axis=0)
    tot2 = jnp.sum(tot * tot)
    counts = jnp.sum(cnt_ref[...], axis=0)
    countsf = counts.astype(jnp.float32)
    present = counts > 0
    k = jnp.sum(present.astype(jnp.int32))
    safe = jnp.where(present, countsf, jnp.float32(1.0))
    T = jnp.sum(rowsq / safe)
    sumsq = jnp.sum(sq_ref[...])
    n = jnp.float32(N)
    bcss = T - tot2 / n
    wcss = sumsq - T
    kf = k.astype(jnp.float32)
    ch = bcss * (n - kf) / ((kf - 1.0) * wcss + jnp.float32(1e-10))
    val = jnp.where((k < 2) | (k == N), jnp.float32(0.0), -ch)
    out_ref[...] = jnp.broadcast_to(val, (1, 1))


def kernel(embeddings, labels):
    labels = labels.reshape(-1)
    mesh = plsc.VectorSubcoreMesh(core_axis_name="c", subcore_axis_name="s")
    part, cnt, sq = pl.kernel(
        _sc_body,
        out_type=(
            jax.ShapeDtypeStruct((2, K, D), jnp.float32),
            jax.ShapeDtypeStruct((NW, K), jnp.int32),
            jax.ShapeDtypeStruct((NW, 16), jnp.float32),
        ),
        mesh=mesh,
        compiler_params=pltpu.CompilerParams(needs_layout_passes=False),
        scratch_types=[
            pltpu.VMEM_SHARED((K, D), jnp.float32),
            pltpu.VMEM((BLK, D), jnp.float32),
            pltpu.VMEM((BLK, D), jnp.float32),
            pltpu.VMEM((BLK, D), jnp.float32),
            pltpu.VMEM((RB,), jnp.int32),
            pltpu.VMEM((RB,), jnp.int32),
            pltpu.VMEM((RB,), jnp.int32),
            pltpu.VMEM((RB,), jnp.int32),
            pltpu.VMEM((RB,), jnp.int32),
            pltpu.VMEM((RB,), jnp.int32),
            pltpu.VMEM((16, K), jnp.int32),
            pltpu.VMEM((K,), jnp.int32),
            pltpu.VMEM((16,), jnp.float32),
            pltpu.SemaphoreType.DMA,
            pltpu.SemaphoreType.DMA,
            pltpu.SemaphoreType.DMA,
            pltpu.SemaphoreType.DMA,
            pltpu.SemaphoreType.DMA,
            pltpu.SemaphoreType.DMA,
            pltpu.SemaphoreType.DMA,
            pltpu.SemaphoreType.DMA,
            pltpu.SemaphoreType.DMA,
        ],
    )(embeddings, labels)
    res = pl.pallas_call(
        _epi_body,
        out_shape=jax.ShapeDtypeStruct((1, 1), jnp.float32),
    )(part, cnt, sq)
    return jnp.reshape(res, ())


# speedup vs baseline: 1.0419x; 1.0154x over previous
import jax
import jax.numpy as jnp
from jax import lax
from jax.experimental import pallas as pl
from jax.experimental.pallas import tpu as pltpu
from jax.experimental.pallas import tpu_sc as plsc

N = 320000
D = 128
K = 1024
RB = 128
BLK = 256
NBLK = N // BLK
NW = 32
BASE_BLKS = NBLK // NW
EXTRA = NBLK - BASE_BLKS * NW
NRING = 3
TRIPS = BASE_BLKS // NRING


def _sc_body(emb, lbl, part_o, cnt_o, sq_o, acc_sp,
             buf0, buf1, buf2, la0, lb0, la1, lb1, la2, lb2, cnt, cntred,
             sqbuf, sin0, sin1, sin2, sl0, sl1, sl2, ssc0, ssc1, ssc2):
    c = lax.axis_index("c")
    s = lax.axis_index("s")
    wid = s * 2 + c

    zf16 = jnp.zeros((16,), jnp.float32)
    zi16 = jnp.zeros((16,), jnp.int32)
    iota16 = lax.broadcasted_iota(jnp.int32, (16,), 0)
    ones16 = jnp.ones((16,), jnp.int32)
    bufs = (buf0, buf1, buf2)
    las = (la0, la1, la2)
    lbs = (lb0, lb1, lb2)
    sins = (sin0, sin1, sin2)
    sls = (sl0, sl1, sl2)
    sscs = (ssc0, ssc1, ssc2)

    def zero_cnt(i, _):
        for j in range(16):
            cnt[j, pl.ds(i * 16, 16)] = zi16
        return 0
    lax.fori_loop(0, K // 16, zero_cnt, 0)
    sqbuf[pl.ds(0, 16)] = zf16

    def zero_buf(i, _):
        for v in range(8):
            buf0[i, pl.ds(v * 16, 16)] = zf16
        return 0
    lax.fori_loop(0, RB, zero_buf, 0)

    pltpu.sync_copy(buf0.at[pl.ds(0, K // 16)],
                    acc_sp.at[pl.ds(s * (K // 16), K // 16)])

    plsc.subcore_barrier()

    start = wid * BASE_BLKS

    def start_in(blkidx, p):
        row0 = blkidx * BLK
        pltpu.async_copy(emb.at[pl.ds(row0, BLK)], bufs[p], sins[p])
        pltpu.async_copy(lbl.at[pl.ds(row0, RB)], las[p], sls[p])
        pltpu.async_copy(lbl.at[pl.ds(row0 + RB, RB)], lbs[p], sls[p])

    def wait_in(p):
        pltpu.make_async_copy(emb.at[pl.ds(0, BLK)], bufs[p], sins[p]).wait()
        pltpu.make_async_copy(lbl.at[pl.ds(0, RB)], las[p], sls[p]).wait()
        pltpu.make_async_copy(lbl.at[pl.ds(0, RB)], lbs[p], sls[p]).wait()

    def start_scat(p):
        da = pltpu.async_copy(bufs[p].at[pl.ds(0, RB)],
                              acc_sp.at[las[p]], sscs[p], add=True)
        db = pltpu.async_copy(bufs[p].at[pl.ds(RB, RB)],
                              acc_sp.at[lbs[p]], sscs[p], add=True)
        return da, db

    def compute(p):
        bf = bufs[p]
        for lb_ref in (las[p], lbs[p]):
            for t in range(8):
                l16 = lb_ref[pl.ds(t * 16, 16)]
                plsc.addupdate_scatter(cnt, [iota16, l16], ones16)

        def srow(r, a):
            for v in range(8):
                x0 = bf[2 * r, pl.ds(v * 16, 16)]
                x1 = bf[2 * r + 1, pl.ds(v * 16, 16)]
                a = a + x0 * x0 + x1 * x1
            return a
        blocksq = lax.fori_loop(0, BLK // 2, srow, jnp.zeros((16,), jnp.float32))
        sqbuf[pl.ds(0, 16)] = sqbuf[pl.ds(0, 16)] + blocksq

    for p in range(NRING):
        start_in(start + p, p)

    def tri_body(i, _):
        b0 = start + NRING * i
        for p in range(NRING):
            wait_in(p)
            d = start_scat(p)
            compute(p)
            d[0].wait()
            d[1].wait()

            @pl.when(i < TRIPS - 1)
            def _():
                start_in(b0 + p + NRING, p)
        return 0

    lax.fori_loop(0, TRIPS, tri_body, 0)

    @pl.when(wid < 2 * EXTRA)
    def _():
        row0 = NW * BASE_BLKS * BLK + wid * RB
        pltpu.sync_copy(emb.at[pl.ds(row0, RB)], buf0.at[pl.ds(0, RB)])
        pltpu.sync_copy(lbl.at[pl.ds(row0, RB)], la0)
        pltpu.sync_copy(buf0.at[pl.ds(0, RB)], acc_sp.at[la0], add=True)
        for t in range(8):
            l16 = la0[pl.ds(t * 16, 16)]
            plsc.addupdate_scatter(cnt, [iota16, l16], ones16)

        def srow2(r, a):
            for v in range(8):
                x0 = buf0[2 * r, pl.ds(v * 16, 16)]
                x1 = buf0[2 * r + 1, pl.ds(v * 16, 16)]
                a = a + x0 * x0 + x1 * x1
            return a
        tailsq = lax.fori_loop(0, RB // 2, srow2, jnp.zeros((16,), jnp.float32))
        sqbuf[pl.ds(0, 16)] = sqbuf[pl.ds(0, 16)] + tailsq

    def red_cnt(g, _):
        a = cnt[0, pl.ds(g * 16, 16)]
        for j in range(1, 16):
            a = a + cnt[j, pl.ds(g * 16, 16)]
        cntred[pl.ds(g * 16, 16)] = a
        return 0
    lax.fori_loop(0, K // 16, red_cnt, 0)

    pltpu.sync_copy(cntred, cnt_o.at[wid])
    pltpu.sync_copy(sqbuf, sq_o.at[wid])

    plsc.subcore_barrier()

    pltpu.sync_copy(acc_sp.at[pl.ds(s * (K // 16), K // 16)],
                    part_o.at[c, pl.ds(s * (K // 16), K // 16)])


def _epi_body(part_ref, cnt_ref, sq_ref, out_ref):
    S = part_ref[0] + part_ref[1]
    rowsq = jnp.sum(S * S, axis=1)
    tot = jnp.sum(S, axis=0)
    tot2 = jnp.sum(tot * tot)
    counts = jnp.sum(cnt_ref[...], axis=0)
    countsf = counts.astype(jnp.float32)
    present = counts > 0
    k = jnp.sum(present.astype(jnp.int32))
    safe = jnp.where(present, countsf, jnp.float32(1.0))
    T = jnp.sum(rowsq / safe)
    sumsq = jnp.sum(sq_ref[...])
    n = jnp.float32(N)
    bcss = T - tot2 / n
    wcss = sumsq - T
    kf = k.astype(jnp.float32)
    ch = bcss * (n - kf) / ((kf - 1.0) * wcss + jnp.float32(1e-10))
    val = jnp.where((k < 2) | (k == N), jnp.float32(0.0), -ch)
    out_ref[...] = jnp.broadcast_to(val, (1, 1))


def kernel(embeddings, labels):
    labels = labels.reshape(-1)
    mesh = plsc.VectorSubcoreMesh(core_axis_name="c", subcore_axis_name="s")
    part, cnt, sq = pl.kernel(
        _sc_body,
        out_type=(
            jax.ShapeDtypeStruct((2, K, D), jnp.float32),
            jax.ShapeDtypeStruct((NW, K), jnp.int32),
            jax.ShapeDtypeStruct((NW, 16), jnp.float32),
        ),
        mesh=mesh,
        compiler_params=pltpu.CompilerParams(needs_layout_passes=False),
        scratch_types=[
            pltpu.VMEM_SHARED((K, D), jnp.float32),
            pltpu.VMEM((BLK, D), jnp.float32),
            pltpu.VMEM((BLK, D), jnp.float32),
            pltpu.VMEM((BLK, D), jnp.float32),
            pltpu.VMEM((RB,), jnp.int32),
            pltpu.VMEM((RB,), jnp.int32),
            pltpu.VMEM((RB,), jnp.int32),
            pltpu.VMEM((RB,), jnp.int32),
            pltpu.VMEM((RB,), jnp.int32),
            pltpu.VMEM((RB,), jnp.int32),
            pltpu.VMEM((16, K), jnp.int32),
            pltpu.VMEM((K,), jnp.int32),
            pltpu.VMEM((16,), jnp.float32),
            pltpu.SemaphoreType.DMA,
            pltpu.SemaphoreType.DMA,
            pltpu.SemaphoreType.DMA,
            pltpu.SemaphoreType.DMA,
            pltpu.SemaphoreType.DMA,
            pltpu.SemaphoreType.DMA,
            pltpu.SemaphoreType.DMA,
            pltpu.SemaphoreType.DMA,
            pltpu.SemaphoreType.DMA,
        ],
    )(embeddings, labels)
    res = pl.pallas_call(
        _epi_body,
        out_shape=jax.ShapeDtypeStruct((1, 1), jnp.float32),
    )(part, cnt, sq)
    return jnp.reshape(res, ())


# speedup vs baseline: 1.0501x; 1.0079x over previous
import jax
import jax.numpy as jnp
from jax import lax
from jax.experimental import pallas as pl
from jax.experimental.pallas import tpu as pltpu
from jax.experimental.pallas import tpu_sc as plsc

N = 320000
D = 128
K = 1024
RB = 128
BLK = 256
NBLK = N // BLK
NW = 32
BASE_BLKS = NBLK // NW
EXTRA = NBLK - BASE_BLKS * NW
NRING = 3
TRIPS = BASE_BLKS // NRING


def _sc_body(emb, lbl, part_o, cnt_o, sq_o, acc_sp,
             buf0, buf1, buf2, la0, lb0, la1, lb1, la2, lb2, cnt, cntred,
             sqbuf, zbuf, sin0, sin1, sin2, sl0, sl1, sl2, ssc0, ssc1, ssc2):
    c = lax.axis_index("c")
    s = lax.axis_index("s")
    wid = s * 2 + c

    zf16 = jnp.zeros((16,), jnp.float32)
    zi16 = jnp.zeros((16,), jnp.int32)
    iota16 = lax.broadcasted_iota(jnp.int32, (16,), 0)
    ones16 = jnp.ones((16,), jnp.int32)
    bufs = (buf0, buf1, buf2)
    las = (la0, la1, la2)
    lbs = (lb0, lb1, lb2)
    sins = (sin0, sin1, sin2)
    sls = (sl0, sl1, sl2)
    sscs = (ssc0, ssc1, ssc2)

    start = wid * BASE_BLKS

    def start_in(blkidx, p):
        row0 = blkidx * BLK
        pltpu.async_copy(emb.at[pl.ds(row0, BLK)], bufs[p], sins[p])
        pltpu.async_copy(lbl.at[pl.ds(row0, RB)], las[p], sls[p])
        pltpu.async_copy(lbl.at[pl.ds(row0 + RB, RB)], lbs[p], sls[p])

    for p in range(NRING):
        start_in(start + p, p)

    def zero_cnt(i, _):
        for j in range(16):
            cnt[j, pl.ds(i * 16, 16)] = zi16
        return 0
    lax.fori_loop(0, K // 16, zero_cnt, 0)
    sqbuf[pl.ds(0, 16)] = zf16

    def zero_zbuf(i, _):
        for v in range(8):
            zbuf[i, pl.ds(v * 16, 16)] = zf16
        return 0
    lax.fori_loop(0, 8, zero_zbuf, 0)

    for j in range(8):
        pltpu.sync_copy(zbuf, acc_sp.at[pl.ds(s * (K // 16) + 8 * j, 8)])

    plsc.subcore_barrier()

    def wait_in(p):
        pltpu.make_async_copy(emb.at[pl.ds(0, BLK)], bufs[p], sins[p]).wait()
        pltpu.make_async_copy(lbl.at[pl.ds(0, RB)], las[p], sls[p]).wait()
        pltpu.make_async_copy(lbl.at[pl.ds(0, RB)], lbs[p], sls[p]).wait()

    def start_scat(p):
        da = pltpu.async_copy(bufs[p].at[pl.ds(0, RB)],
                              acc_sp.at[las[p]], sscs[p], add=True)
        db = pltpu.async_copy(bufs[p].at[pl.ds(RB, RB)],
                              acc_sp.at[lbs[p]], sscs[p], add=True)
        return da, db

    def compute(p):
        bf = bufs[p]
        for lb_ref in (las[p], lbs[p]):
            for t in range(8):
                l16 = lb_ref[pl.ds(t * 16, 16)]
                plsc.addupdate_scatter(cnt, [iota16, l16], ones16)

        def srow(r, a):
            for v in range(8):
                x0 = bf[2 * r, pl.ds(v * 16, 16)]
                x1 = bf[2 * r + 1, pl.ds(v * 16, 16)]
                a = a + x0 * x0 + x1 * x1
            return a
        blocksq = lax.fori_loop(0, BLK // 2, srow, jnp.zeros((16,), jnp.float32))
        sqbuf[pl.ds(0, 16)] = sqbuf[pl.ds(0, 16)] + blocksq

    def tri_body(i, _):
        b0 = start + NRING * i
        for p in range(NRING):
            wait_in(p)
            d = start_scat(p)
            compute(p)
            d[0].wait()
            d[1].wait()

            @pl.when(i < TRIPS - 1)
            def _():
                start_in(b0 + p + NRING, p)
        return 0

    lax.fori_loop(0, TRIPS, tri_body, 0)

    @pl.when(wid < 2 * EXTRA)
    def _():
        row0 = NW * BASE_BLKS * BLK + wid * RB
        pltpu.sync_copy(emb.at[pl.ds(row0, RB)], buf0.at[pl.ds(0, RB)])
        pltpu.sync_copy(lbl.at[pl.ds(row0, RB)], la0)
        pltpu.sync_copy(buf0.at[pl.ds(0, RB)], acc_sp.at[la0], add=True)
        for t in range(8):
            l16 = la0[pl.ds(t * 16, 16)]
            plsc.addupdate_scatter(cnt, [iota16, l16], ones16)

        def srow2(r, a):
            for v in range(8):
                x0 = buf0[2 * r, pl.ds(v * 16, 16)]
                x1 = buf0[2 * r + 1, pl.ds(v * 16, 16)]
                a = a + x0 * x0 + x1 * x1
            return a
        tailsq = lax.fori_loop(0, RB // 2, srow2, jnp.zeros((16,), jnp.float32))
        sqbuf[pl.ds(0, 16)] = sqbuf[pl.ds(0, 16)] + tailsq

    def red_cnt(g, _):
        a = cnt[0, pl.ds(g * 16, 16)]
        for j in range(1, 16):
            a = a + cnt[j, pl.ds(g * 16, 16)]
        cntred[pl.ds(g * 16, 16)] = a
        return 0
    lax.fori_loop(0, K // 16, red_cnt, 0)

    pltpu.sync_copy(cntred, cnt_o.at[wid])
    pltpu.sync_copy(sqbuf, sq_o.at[wid])

    plsc.subcore_barrier()

    pltpu.sync_copy(acc_sp.at[pl.ds(s * (K // 16), K // 16)],
                    part_o.at[c, pl.ds(s * (K // 16), K // 16)])


def _epi_body(part_ref, cnt_ref, sq_ref, out_ref):
    S = part_ref[0] + part_ref[1]
    rowsq = jnp.sum(S * S, axis=1)
    tot = jnp.sum(S, axis=0)
    tot2 = jnp.sum(tot * tot)
    counts = jnp.sum(cnt_ref[...], axis=0)
    countsf = counts.astype(jnp.float32)
    present = counts > 0
    k = jnp.sum(present.astype(jnp.int32))
    safe = jnp.where(present, countsf, jnp.float32(1.0))
    T = jnp.sum(rowsq / safe)
    sumsq = jnp.sum(sq_ref[...])
    n = jnp.float32(N)
    bcss = T - tot2 / n
    wcss = sumsq - T
    kf = k.astype(jnp.float32)
    ch = bcss * (n - kf) / ((kf - 1.0) * wcss + jnp.float32(1e-10))
    val = jnp.where((k < 2) | (k == N), jnp.float32(0.0), -ch)
    out_ref[...] = jnp.broadcast_to(val, (1, 1))


def kernel(embeddings, labels):
    labels = labels.reshape(-1)
    mesh = plsc.VectorSubcoreMesh(core_axis_name="c", subcore_axis_name="s")
    part, cnt, sq = pl.kernel(
        _sc_body,
        out_type=(
            jax.ShapeDtypeStruct((2, K, D), jnp.float32),
            jax.ShapeDtypeStruct((NW, K), jnp.int32),
            jax.ShapeDtypeStruct((NW, 16), jnp.float32),
        ),
        mesh=mesh,
        compiler_params=pltpu.CompilerParams(needs_layout_passes=False),
        scratch_types=[
            pltpu.VMEM_SHARED((K, D), jnp.float32),
            pltpu.VMEM((BLK, D), jnp.float32),
            pltpu.VMEM((BLK, D), jnp.float32),
            pltpu.VMEM((BLK, D), jnp.float32),
            pltpu.VMEM((RB,), jnp.int32),
            pltpu.VMEM((RB,), jnp.int32),
            pltpu.VMEM((RB,), jnp.int32),
            pltpu.VMEM((RB,), jnp.int32),
            pltpu.VMEM((RB,), jnp.int32),
            pltpu.VMEM((RB,), jnp.int32),
            pltpu.VMEM((16, K), jnp.int32),
            pltpu.VMEM((K,), jnp.int32),
            pltpu.VMEM((16,), jnp.float32),
            pltpu.VMEM((8, D), jnp.float32),
            pltpu.SemaphoreType.DMA,
            pltpu.SemaphoreType.DMA,
            pltpu.SemaphoreType.DMA,
            pltpu.SemaphoreType.DMA,
            pltpu.SemaphoreType.DMA,
            pltpu.SemaphoreType.DMA,
            pltpu.SemaphoreType.DMA,
            pltpu.SemaphoreType.DMA,
            pltpu.SemaphoreType.DMA,
        ],
    )(embeddings, labels)
    res = pl.pallas_call(
        _epi_body,
        out_shape=jax.ShapeDtypeStruct((1, 1), jnp.float32),
    )(part, cnt, sq)
    return jnp.reshape(res, ())
